# baseline (speedup 1.0000x reference)
import jax
import jax.numpy as jnp
from jax import lax
from jax.experimental import pallas as pl
from jax.experimental.pallas import tpu as pltpu

N, K, M = 16384, 256, 64
BLK = 8192
NBLK = N // BLK


def _fused_kernel(x_ref, c_ref, mind_ref, idx_ref,
                  dt_ref, dn_ref, rmin_ref, cixv_ref, cixs_ref,
                  pick_ref, sem):
    i = pl.program_id(0)
    xb = jnp.transpose(x_ref[...])
    c = c_ref[...]
    xcT = lax.dot_general(c, xb, (((1,), (0,)), ((), ())),
                          preferred_element_type=jnp.float32,
                          precision=lax.Precision.HIGHEST)
    xxT = jnp.sum(xb * xb, axis=0, keepdims=True)
    cc = jnp.sum(c * c, axis=1, keepdims=True)
    dT = jnp.sqrt(jnp.maximum(cc + xxT - 2.0 * xcT, 0.0))
    dt_ref[i] = dT
    mind_ref[...] = jnp.min(dT, axis=0, keepdims=True)

    dn = jnp.transpose(dT)
    dn_ref[i] = dn
    bm = dn
    h = BLK // 2
    while h >= 8:
        bm = jnp.minimum(bm[:h], bm[h:])
        h //= 2
    rmin_ref[...] = jnp.where(i == 0, bm, jnp.minimum(rmin_ref[...], bm))

    @pl.when(i == NBLK - 1)
    def _greedy():
        i0 = lax.broadcasted_iota(jnp.int32, (NBLK, BLK), 0)
        i1 = lax.broadcasted_iota(jnp.int32, (NBLK, BLK), 1)
        iota = i0 * BLK + i1
        col_iota = lax.broadcasted_iota(jnp.int32, (1, M), 1)
        big = jnp.int32(N)
        inf = jnp.float32(jnp.inf)

        def vreg_min_11(a):
            h = NBLK
            while h > 8:
                a = jnp.minimum(a[:h // 2], a[h // 2:])
                h //= 2
            w = BLK // 2
            while w >= 128:
                a = jnp.minimum(a[:, :w], a[:, w:])
                w //= 2
            return jnp.min(a, keepdims=True)

        mcol = rmin_ref[...]
        for h2 in (4, 2, 1):
            mcol = jnp.minimum(mcol[:h2], mcol[h2:])
        riota = lax.broadcasted_iota(jnp.int32, (BLK, M), 0)
        ci8 = None
        for b in range(NBLK):
            cnd = jnp.where(dn_ref[b] == mcol, riota + b * BLK, big)
            h2 = BLK // 2
            while h2 >= 8:
                cnd = jnp.minimum(cnd[:h2], cnd[h2:])
                h2 //= 2
            ci8 = cnd if ci8 is None else jnp.minimum(ci8, cnd)
        for h2 in (4, 2, 1):
            ci8 = jnp.minimum(ci8[:h2], ci8[h2:])
        cixv_ref[...] = ci8
        pltpu.make_async_copy(cixv_ref, cixs_ref, sem).start()
        pltpu.make_async_copy(cixv_ref, cixs_ref, sem).wait()

        penalty = jnp.zeros((NBLK, BLK), jnp.float32)
        idxs = jnp.zeros((1, M), jnp.int32)
        taken = []
        for j in range(M):
            cj = cixs_ref[0, j]
            pick_ref[0, 0] = cj
            if taken:
                inv = cj == taken[0]
                for t in taken[1:]:
                    inv = inv | (cj == t)

                @pl.when(inv)
                def _fallback():
                    masked = dt_ref[:, j, :] + penalty
                    mv = vreg_min_11(masked)
                    cand = jnp.where(masked == mv, iota, big)
                    ixa = vreg_min_11(cand)
                    pick_ref[0, 0] = ixa[0, 0]

            ix_s = pick_ref[0, 0]
            taken.append(ix_s)
            penalty = jnp.where(iota == ix_s, inf, penalty)
            idxs = jnp.where(col_iota == j, ix_s, idxs)
        idx_ref[...] = idxs


_fused_call = pl.pallas_call(
    _fused_kernel,
    grid=(NBLK,),
    in_specs=[pl.BlockSpec((BLK, K), lambda i: (i, 0)),
              pl.BlockSpec((M, K), lambda i: (0, 0))],
    out_specs=[pl.BlockSpec((1, BLK), lambda i: (0, i)),
               pl.BlockSpec((1, M), lambda i: (0, 0))],
    out_shape=[jax.ShapeDtypeStruct((1, N), jnp.float32),
               jax.ShapeDtypeStruct((1, M), jnp.int32)],
    scratch_shapes=[pltpu.VMEM((NBLK, M, BLK), jnp.float32),
                    pltpu.VMEM((NBLK, BLK, M), jnp.float32),
                    pltpu.VMEM((8, M), jnp.float32),
                    pltpu.VMEM((1, M), jnp.int32),
                    pltpu.SMEM((1, M), jnp.int32),
                    pltpu.SMEM((1, 1), jnp.int32),
                    pltpu.SemaphoreType.DMA],
)


def kernel(x, new_centers):
    mind, idxs = _fused_call(x, new_centers)
    return (idxs.reshape(M).astype(jnp.int64), mind.reshape(N), new_centers)


# speedup vs baseline: 1.0414x; 1.0414x over previous
import jax
import jax.numpy as jnp
from jax import lax
from jax.experimental import pallas as pl
from jax.experimental.pallas import tpu as pltpu

N, K, M = 16384, 256, 64
BLK = 4096
NBLK = N // BLK


def _fused_kernel(x_ref, c_ref, mind_ref, idx_ref,
                  dt_ref, dn_ref, rmin_ref, cixv_ref, cixs_ref,
                  pick_ref, sem):
    i = pl.program_id(0)
    xb = jnp.transpose(x_ref[...])
    c = c_ref[...]
    xcT = lax.dot_general(c, xb, (((1,), (0,)), ((), ())),
                          preferred_element_type=jnp.float32,
                          precision=lax.Precision.HIGHEST)
    xxT = jnp.sum(xb * xb, axis=0, keepdims=True)
    cc = jnp.sum(c * c, axis=1, keepdims=True)
    dT = jnp.sqrt(jnp.maximum(cc + xxT - 2.0 * xcT, 0.0))
    dt_ref[i] = dT
    mind_ref[...] = jnp.min(dT, axis=0, keepdims=True)

    dn = jnp.transpose(dT)
    dn_ref[i] = dn
    bm = dn
    h = BLK // 2
    while h >= 8:
        bm = jnp.minimum(bm[:h], bm[h:])
        h //= 2
    rmin_ref[...] = jnp.where(i == 0, bm, jnp.minimum(rmin_ref[...], bm))

    @pl.when(i == NBLK - 1)
    def _greedy():
        i0 = lax.broadcasted_iota(jnp.int32, (NBLK, BLK), 0)
        i1 = lax.broadcasted_iota(jnp.int32, (NBLK, BLK), 1)
        iota = i0 * BLK + i1
        col_iota = lax.broadcasted_iota(jnp.int32, (1, M), 1)
        big = jnp.int32(N)
        inf = jnp.float32(jnp.inf)

        def vreg_min_11(a):
            h = NBLK
            while h > 8:
                a = jnp.minimum(a[:h // 2], a[h // 2:])
                h //= 2
            w = BLK // 2
            while w >= 128:
                a = jnp.minimum(a[:, :w], a[:, w:])
                w //= 2
            return jnp.min(a, keepdims=True)

        mcol = rmin_ref[...]
        for h2 in (4, 2, 1):
            mcol = jnp.minimum(mcol[:h2], mcol[h2:])
        riota = lax.broadcasted_iota(jnp.int32, (BLK, M), 0)
        ci8 = None
        for b in range(NBLK):
            cnd = jnp.where(dn_ref[b] == mcol, riota + b * BLK, big)
            h2 = BLK // 2
            while h2 >= 8:
                cnd = jnp.minimum(cnd[:h2], cnd[h2:])
                h2 //= 2
            ci8 = cnd if ci8 is None else jnp.minimum(ci8, cnd)
        for h2 in (4, 2, 1):
            ci8 = jnp.minimum(ci8[:h2], ci8[h2:])
        cixv_ref[...] = ci8
        pltpu.make_async_copy(cixv_ref, cixs_ref, sem).start()
        pltpu.make_async_copy(cixv_ref, cixs_ref, sem).wait()

        penalty = jnp.zeros((NBLK, BLK), jnp.float32)
        idxs = jnp.zeros((1, M), jnp.int32)
        taken = []
        for j in range(M):
            cj = cixs_ref[0, j]
            pick_ref[0, 0] = cj
            if taken:
                inv = cj == taken[0]
                for t in taken[1:]:
                    inv = inv | (cj == t)

                @pl.when(inv)
                def _fallback():
                    masked = dt_ref[:, j, :] + penalty
                    mv = vreg_min_11(masked)
                    cand = jnp.where(masked == mv, iota, big)
                    ixa = vreg_min_11(cand)
                    pick_ref[0, 0] = ixa[0, 0]

            ix_s = pick_ref[0, 0]
            taken.append(ix_s)
            penalty = jnp.where(iota == ix_s, inf, penalty)
            idxs = jnp.where(col_iota == j, ix_s, idxs)
        idx_ref[...] = idxs


_fused_call = pl.pallas_call(
    _fused_kernel,
    grid=(NBLK,),
    in_specs=[pl.BlockSpec((BLK, K), lambda i: (i, 0)),
              pl.BlockSpec((M, K), lambda i: (0, 0))],
    out_specs=[pl.BlockSpec((1, BLK), lambda i: (0, i)),
               pl.BlockSpec((1, M), lambda i: (0, 0))],
    out_shape=[jax.ShapeDtypeStruct((1, N), jnp.float32),
               jax.ShapeDtypeStruct((1, M), jnp.int32)],
    scratch_shapes=[pltpu.VMEM((NBLK, M, BLK), jnp.float32),
                    pltpu.VMEM((NBLK, BLK, M), jnp.float32),
                    pltpu.VMEM((8, M), jnp.float32),
                    pltpu.VMEM((1, M), jnp.int32),
                    pltpu.SMEM((1, M), jnp.int32),
                    pltpu.SMEM((1, 1), jnp.int32),
                    pltpu.SemaphoreType.DMA],
)


def kernel(x, new_centers):
    mind, idxs = _fused_call(x, new_centers)
    return (idxs.reshape(M).astype(jnp.int64), mind.reshape(N), new_centers)
